# Initial kernel scaffold; baseline (speedup 1.0000x reference)
import functools

import jax
import jax.numpy as jnp
from jax import lax
from jax.experimental import pallas as pl
from jax.experimental.pallas import tpu as pltpu
from jax.experimental.pallas import tpu_sc as plsc

N = 10000
NPAD = 10240
E = 320000
DIN = 128
DHID = 256
DOUT = 128

NC = 2
NS = 16
L = 16
NW = NC * NS

EROWS = 2528
EPAD = EROWS * 128
RPT = EROWS // NS
NROWS_PT = NPAD // NS


def _sc_mesh():
    return plsc.VectorSubcoreMesh(
        core_axis_name="c", subcore_axis_name="s", num_cores=NC, num_subcores=NS
    )


_DEG_CH = 400
_EPW = E // NW


def _deg_partials(dst_flat):
    @functools.partial(
        pl.kernel,
        mesh=_sc_mesh(),
        out_type=jax.ShapeDtypeStruct((NC * NPAD,), jnp.float32),
        scratch_types=[
            pltpu.VMEM((_DEG_CH,), jnp.int32),
            pltpu.VMEM((NPAD,), jnp.float32),
            pltpu.VMEM((NROWS_PT,), jnp.float32),
            pltpu.VMEM((NROWS_PT,), jnp.float32),
            pltpu.VMEM_SHARED((NS, NPAD), jnp.float32),
        ],
    )
    def deg_kernel(dst_hbm, out_hbm, idx_v, deg_v, acc_v, tmp_v, sh):
        cid = lax.axis_index("c")
        sid = lax.axis_index("s")
        wid = sid * NC + cid

        def zero_deg(i, _):
            deg_v[pl.ds(i * L, L)] = jnp.zeros((L,), jnp.float32)
            return 0

        lax.fori_loop(0, NPAD // L, zero_deg, 0)

        ones = jnp.ones((L,), jnp.float32)
        base = wid * _EPW

        def chunk(j, _):
            pltpu.sync_copy(dst_hbm.at[pl.ds(base + j * _DEG_CH, _DEG_CH)], idx_v)

            def scat(k, _):
                idx = idx_v[pl.ds(k * L, L)]
                plsc.addupdate_scatter(deg_v, [idx], ones)
                return 0

            lax.fori_loop(0, _DEG_CH // L, scat, 0)
            return 0

        lax.fori_loop(0, _EPW // _DEG_CH, chunk, 0)

        pltpu.sync_copy(deg_v, sh.at[sid])
        plsc.subcore_barrier()

        cbase = sid * NROWS_PT

        def zero_acc(i, _):
            acc_v[pl.ds(i * L, L)] = jnp.zeros((L,), jnp.float32)
            return 0

        lax.fori_loop(0, NROWS_PT // L, zero_acc, 0)

        def red(a, _):
            pltpu.sync_copy(sh.at[a, pl.ds(cbase, NROWS_PT)], tmp_v)

            def add(k, _):
                sl = pl.ds(k * L, L)
                acc_v[sl] = acc_v[sl] + tmp_v[sl]
                return 0

            lax.fori_loop(0, NROWS_PT // L, add, 0)
            return 0

        lax.fori_loop(0, NS, red, 0)
        pltpu.sync_copy(acc_v, out_hbm.at[pl.ds(cid * NPAD + cbase, NROWS_PT)])

    return deg_kernel(dst_flat)


def _make_agg(dh):
    @functools.partial(
        pl.kernel,
        mesh=_sc_mesh(),
        out_type=jax.ShapeDtypeStruct((NC * NPAD, dh), jnp.float32),
        scratch_types=[
            pltpu.VMEM((RPT, 128), jnp.int32),
            pltpu.VMEM((RPT, 128), jnp.int32),
            pltpu.VMEM((128, dh), jnp.float32),
            pltpu.VMEM_SHARED((NPAD, dh), jnp.float32),
            pltpu.SemaphoreType.DMA,
        ],
    )
    def agg_kernel(h_hbm, src_hbm, dst_hbm, out_hbm, si_v, di_v, rows_v, agg_sh, sem):
        cid = lax.axis_index("c")
        sid = lax.axis_index("s")
        rbase = sid * RPT

        pltpu.sync_copy(src_hbm.at[pl.ds(rbase, RPT)], si_v)
        pltpu.sync_copy(dst_hbm.at[pl.ds(rbase, RPT)], di_v)

        off = cid * NPAD

        def off_row(i, _):
            def off_vec(k, _):
                sl = pl.ds(k * L, L)
                si_v[i, sl] = si_v[i, sl] + off
                return 0

            lax.fori_loop(0, 128 // L, off_vec, 0)
            return 0

        lax.fori_loop(0, RPT, off_row, 0)

        def zrow(i, _):
            def zvec(k, _):
                rows_v[i, pl.ds(k * L, L)] = jnp.zeros((L,), jnp.float32)
                return 0

            lax.fori_loop(0, dh // L, zvec, 0)
            return 0

        lax.fori_loop(0, 128, zrow, 0)

        def clr(k, _):
            pltpu.sync_copy(rows_v, agg_sh.at[pl.ds(sid * NROWS_PT + k * 128, 128)])
            return 0

        lax.fori_loop(0, NROWS_PT // 128, clr, 0)
        plsc.subcore_barrier()

        def main(j, _):
            pltpu.async_copy(h_hbm.at[si_v.at[j]], rows_v, sem).wait()
            pltpu.sync_copy(rows_v, agg_sh.at[di_v.at[j]], add=True)
            return 0

        lax.fori_loop(0, RPT, main, 0)
        plsc.subcore_barrier()

        pltpu.sync_copy(
            agg_sh.at[pl.ds(sid * NROWS_PT, NROWS_PT)],
            out_hbm.at[pl.ds(cid * NPAD + sid * NROWS_PT, NROWS_PT)],
        )

    return agg_kernel


_agg128 = _make_agg(DHID // NC)
_agg64 = _make_agg(DOUT // NC)


_BN = 512


def _dis(deg_ref):
    return lax.rsqrt(deg_ref[:, 0:1] + deg_ref[:, 1:2] + 1.0)


def _tc1_body(x_ref, w_ref, deg_ref, out_ref):
    h = jnp.dot(x_ref[...], w_ref[...], preferred_element_type=jnp.float32)
    out_ref[0] = _dis(deg_ref) * h


def _tc1(x_pad, w1, deg_t):
    return pl.pallas_call(
        _tc1_body,
        grid=(NC, NPAD // _BN),
        in_specs=[
            pl.BlockSpec((_BN, DIN), lambda c, i: (i, 0)),
            pl.BlockSpec((DIN, DHID // NC), lambda c, i: (0, c)),
            pl.BlockSpec((_BN, 2), lambda c, i: (i, 0)),
        ],
        out_specs=pl.BlockSpec((1, _BN, DHID // NC), lambda c, i: (c, i, 0)),
        out_shape=jax.ShapeDtypeStruct((NC, NPAD, DHID // NC), jnp.float32),
    )(x_pad, w1, deg_t)


def _tc2_body(agg_ref, hp_ref, deg_ref, b1_ref, w2_ref, out_ref):
    dis = _dis(deg_ref)
    b = b1_ref[...]
    za = jnp.maximum(dis * (agg_ref[0] + hp_ref[0]) + b[:, :128], 0.0)
    zb = jnp.maximum(dis * (agg_ref[1] + hp_ref[1]) + b[:, 128:], 0.0)
    w = w2_ref[...]
    h2 = jnp.dot(za, w[:128], preferred_element_type=jnp.float32)
    h2 = h2 + jnp.dot(zb, w[128:], preferred_element_type=jnp.float32)
    h2 = dis * h2
    out_ref[0] = h2[:, :64]
    out_ref[1] = h2[:, 64:]


def _tc2(agg1, h1p, deg_t, b1, w2):
    return pl.pallas_call(
        _tc2_body,
        grid=(NPAD // _BN,),
        in_specs=[
            pl.BlockSpec((NC, _BN, DHID // NC), lambda i: (0, i, 0)),
            pl.BlockSpec((NC, _BN, DHID // NC), lambda i: (0, i, 0)),
            pl.BlockSpec((_BN, 2), lambda i: (i, 0)),
            pl.BlockSpec((1, DHID), lambda i: (0, 0)),
            pl.BlockSpec((DHID, DOUT), lambda i: (0, 0)),
        ],
        out_specs=pl.BlockSpec((NC, _BN, DOUT // NC), lambda i: (0, i, 0)),
        out_shape=jax.ShapeDtypeStruct((NC, NPAD, DOUT // NC), jnp.float32),
    )(agg1, h1p, deg_t, b1, w2)


def _tc3_body(agg_ref, hp_ref, deg_ref, b2_ref, out_ref):
    dis = _dis(deg_ref)
    b = b2_ref[...]
    ya = jnp.maximum(dis * (agg_ref[0] + hp_ref[0]) + b[:, :64], 0.0)
    yb = jnp.maximum(dis * (agg_ref[1] + hp_ref[1]) + b[:, 64:], 0.0)
    out_ref[...] = jnp.concatenate([ya, yb], axis=1)


def _tc3(agg2, h2p, deg_t, b2):
    return pl.pallas_call(
        _tc3_body,
        grid=(NPAD // _BN,),
        in_specs=[
            pl.BlockSpec((NC, _BN, DOUT // NC), lambda i: (0, i, 0)),
            pl.BlockSpec((NC, _BN, DOUT // NC), lambda i: (0, i, 0)),
            pl.BlockSpec((_BN, 2), lambda i: (i, 0)),
            pl.BlockSpec((1, DOUT), lambda i: (0, 0)),
        ],
        out_specs=pl.BlockSpec((_BN, DOUT), lambda i: (i, 0)),
        out_shape=jax.ShapeDtypeStruct((NPAD, DOUT), jnp.float32),
    )(agg2, h2p, deg_t, b2)


def kernel(x, edge_index, W1, b1, W2, b2):
    src = edge_index[0]
    dst = edge_index[1]

    deg2 = _deg_partials(dst)
    deg_t = deg2.reshape(NC, NPAD).T

    pad = jnp.full((EPAD - E,), N, dtype=src.dtype)
    src2d = jnp.concatenate([src, pad]).reshape(EROWS, 128)
    dst2d = jnp.concatenate([dst, pad]).reshape(EROWS, 128)

    x_pad = jnp.zeros((NPAD, DIN), x.dtype).at[:N].set(x)

    h1p = _tc1(x_pad, W1, deg_t)
    agg1 = _agg128(h1p.reshape(NC * NPAD, DHID // NC), src2d, dst2d)
    agg1 = agg1.reshape(NC, NPAD, DHID // NC)
    h2p = _tc2(agg1, h1p, deg_t, b1.reshape(1, DHID), W2)
    agg2 = _agg64(h2p.reshape(NC * NPAD, DOUT // NC), src2d, dst2d)
    agg2 = agg2.reshape(NC, NPAD, DOUT // NC)
    out = _tc3(agg2, h2p, deg_t, b2.reshape(1, DOUT))
    return out[:N]


# speedup vs baseline: 7.6160x; 7.6160x over previous
import functools

import jax
import jax.numpy as jnp
from jax import lax
from jax.experimental import pallas as pl
from jax.experimental.pallas import tpu as pltpu
from jax.experimental.pallas import tpu_sc as plsc

N = 10000
NPAD = 10240
E = 320000
DIN = 128
DHID = 256
DOUT = 128

NC = 2
NS = 16
L = 16
NW = NC * NS

EROWS = 2560
EPAD = EROWS * 128
RPT = EROWS // NS
NROWS_PT = NPAD // NS


def _sc_mesh():
    return plsc.VectorSubcoreMesh(
        core_axis_name="c", subcore_axis_name="s", num_cores=NC, num_subcores=NS
    )


_RPW = EROWS // NW


def _deg_partials(dst2d):
    @functools.partial(
        pl.kernel,
        mesh=_sc_mesh(),
        out_type=jax.ShapeDtypeStruct((NC * NPAD, 128), jnp.float32),
        scratch_types=[
            pltpu.VMEM((_RPW, 128), jnp.int32),
            pltpu.VMEM((128, 128), jnp.float32),
            pltpu.VMEM_SHARED((NPAD, 128), jnp.float32),
        ],
    )
    def deg_kernel(dst_hbm, out_hbm, di_v, rows_v, deg_sh):
        cid = lax.axis_index("c")
        sid = lax.axis_index("s")
        rbase = (cid * NS + sid) * _RPW
        obase = sid * NROWS_PT

        def zrow(i, _):
            def zvec(k, _):
                rows_v[i, pl.ds(k * L, L)] = jnp.zeros((L,), jnp.float32)
                return 0

            lax.fori_loop(0, 128 // L, zvec, 0)
            return 0

        lax.fori_loop(0, 128, zrow, 0)

        def clr(k, _):
            pltpu.sync_copy(rows_v, deg_sh.at[pl.ds(obase + k * 128, 128)])
            return 0

        lax.fori_loop(0, NROWS_PT // 128, clr, 0)
        plsc.subcore_barrier()

        def set1(i, _):
            def svec(k, _):
                rows_v[i, pl.ds(k * L, L)] = jnp.ones((L,), jnp.float32)
                return 0

            lax.fori_loop(0, 128 // L, svec, 0)
            return 0

        lax.fori_loop(0, 128, set1, 0)

        pltpu.sync_copy(dst_hbm.at[pl.ds(rbase, _RPW)], di_v)

        def main(j, _):
            pltpu.sync_copy(rows_v, deg_sh.at[di_v.at[j]], add=True)
            return 0

        lax.fori_loop(0, _RPW, main, 0)
        plsc.subcore_barrier()

        pltpu.sync_copy(deg_sh.at[pl.ds(obase, NROWS_PT)],
                        out_hbm.at[pl.ds(cid * NPAD + obase, NROWS_PT)])

    return deg_kernel(dst2d)


CHR = 16


def _make_agg(dh):
    @functools.partial(
        pl.kernel,
        mesh=_sc_mesh(),
        out_type=jax.ShapeDtypeStruct((NC * NPAD, dh), jnp.float32),
        scratch_types=[
            pltpu.VMEM((CHR, 128), jnp.int32),
            pltpu.VMEM((CHR, 128), jnp.int32),
            pltpu.VMEM((128, dh), jnp.float32),
            pltpu.VMEM_SHARED((NPAD, dh), jnp.float32),
            pltpu.SemaphoreType.DMA,
        ],
    )
    def agg_kernel(h_hbm, src_hbm, dst_hbm, out_hbm, si_v, di_v, rows_v, agg_sh, sem):
        cid = lax.axis_index("c")
        sid = lax.axis_index("s")
        rbase = sid * RPT
        off = cid * NPAD

        def zrow(i, _):
            def zvec(k, _):
                rows_v[i, pl.ds(k * L, L)] = jnp.zeros((L,), jnp.float32)
                return 0

            lax.fori_loop(0, dh // L, zvec, 0)
            return 0

        lax.fori_loop(0, 128, zrow, 0)

        def clr(k, _):
            pltpu.sync_copy(rows_v, agg_sh.at[pl.ds(sid * NROWS_PT + k * 128, 128)])
            return 0

        lax.fori_loop(0, NROWS_PT // 128, clr, 0)
        plsc.subcore_barrier()

        def chunk(q, _):
            rq = rbase + q * CHR
            pltpu.sync_copy(src_hbm.at[pl.ds(rq, CHR)], si_v)
            pltpu.sync_copy(dst_hbm.at[pl.ds(rq, CHR)], di_v)

            def off_row(i, _):
                def off_vec(k, _):
                    sl = pl.ds(k * L, L)
                    si_v[i, sl] = si_v[i, sl] + off
                    return 0

                lax.fori_loop(0, 128 // L, off_vec, 0)
                return 0

            lax.fori_loop(0, CHR, off_row, 0)

            def main(j, _):
                pltpu.async_copy(h_hbm.at[si_v.at[j]], rows_v, sem).wait()
                pltpu.sync_copy(rows_v, agg_sh.at[di_v.at[j]], add=True)
                return 0

            lax.fori_loop(0, CHR, main, 0)
            return 0

        lax.fori_loop(0, RPT // CHR, chunk, 0)
        plsc.subcore_barrier()

        pltpu.sync_copy(
            agg_sh.at[pl.ds(sid * NROWS_PT, NROWS_PT)],
            out_hbm.at[pl.ds(cid * NPAD + sid * NROWS_PT, NROWS_PT)],
        )

    return agg_kernel


_agg128 = _make_agg(DHID // NC)


_RPT_ES = EROWS // NW


def _agg_edge(h, src2d, dst2d):
    @functools.partial(
        pl.kernel,
        mesh=_sc_mesh(),
        out_type=jax.ShapeDtypeStruct((NC * NPAD, DOUT), jnp.float32),
        scratch_types=[
            pltpu.VMEM((CHR, 128), jnp.int32),
            pltpu.VMEM((CHR, 128), jnp.int32),
            pltpu.VMEM((128, DOUT), jnp.float32),
            pltpu.VMEM_SHARED((NPAD, DOUT), jnp.float32),
            pltpu.SemaphoreType.DMA,
        ],
    )
    def agg_kernel(h_hbm, src_hbm, dst_hbm, out_hbm, si_v, di_v, rows_v, agg_sh, sem):
        cid = lax.axis_index("c")
        sid = lax.axis_index("s")
        rbase = (cid * NS + sid) * _RPT_ES

        def zrow(i, _):
            def zvec(k, _):
                rows_v[i, pl.ds(k * L, L)] = jnp.zeros((L,), jnp.float32)
                return 0

            lax.fori_loop(0, DOUT // L, zvec, 0)
            return 0

        lax.fori_loop(0, 128, zrow, 0)

        def clr(k, _):
            pltpu.sync_copy(rows_v, agg_sh.at[pl.ds(sid * NROWS_PT + k * 128, 128)])
            return 0

        lax.fori_loop(0, NROWS_PT // 128, clr, 0)
        plsc.subcore_barrier()

        def chunk(q, _):
            rq = rbase + q * CHR
            pltpu.sync_copy(src_hbm.at[pl.ds(rq, CHR)], si_v)
            pltpu.sync_copy(dst_hbm.at[pl.ds(rq, CHR)], di_v)

            def main(j, _):
                pltpu.async_copy(h_hbm.at[si_v.at[j]], rows_v, sem).wait()
                pltpu.sync_copy(rows_v, agg_sh.at[di_v.at[j]], add=True)
                return 0

            lax.fori_loop(0, CHR, main, 0)
            return 0

        lax.fori_loop(0, _RPT_ES // CHR, chunk, 0)
        plsc.subcore_barrier()

        pltpu.sync_copy(
            agg_sh.at[pl.ds(sid * NROWS_PT, NROWS_PT)],
            out_hbm.at[pl.ds(cid * NPAD + sid * NROWS_PT, NROWS_PT)],
        )

    return agg_kernel(h, src2d, dst2d)


_BN = 512


def _dis(deg_ref):
    return lax.rsqrt(deg_ref[:, 0:1] + deg_ref[:, 1:2] + 1.0)


def _tc1_body(x_ref, w_ref, deg_ref, out_ref):
    h = jnp.dot(x_ref[...], w_ref[...], preferred_element_type=jnp.float32)
    out_ref[0] = _dis(deg_ref) * h


def _tc1(x_pad, w1, deg_t):
    return pl.pallas_call(
        _tc1_body,
        grid=(NC, NPAD // _BN),
        in_specs=[
            pl.BlockSpec((_BN, DIN), lambda c, i: (i, 0)),
            pl.BlockSpec((DIN, DHID // NC), lambda c, i: (0, c)),
            pl.BlockSpec((_BN, 2), lambda c, i: (i, 0)),
        ],
        out_specs=pl.BlockSpec((1, _BN, DHID // NC), lambda c, i: (c, i, 0)),
        out_shape=jax.ShapeDtypeStruct((NC, NPAD, DHID // NC), jnp.float32),
    )(x_pad, w1, deg_t)


def _tc2_body(agg_ref, hp_ref, deg_ref, b1_ref, w2_ref, out_ref):
    dis = _dis(deg_ref)
    b = b1_ref[...]
    za = jnp.maximum(dis * (agg_ref[0] + hp_ref[0]) + b[:, :128], 0.0)
    zb = jnp.maximum(dis * (agg_ref[1] + hp_ref[1]) + b[:, 128:], 0.0)
    w = w2_ref[...]
    h2 = jnp.dot(za, w[:128], preferred_element_type=jnp.float32)
    h2 = h2 + jnp.dot(zb, w[128:], preferred_element_type=jnp.float32)
    out_ref[...] = dis * h2


def _tc2(agg1, h1p, deg_t, b1, w2):
    return pl.pallas_call(
        _tc2_body,
        grid=(NPAD // _BN,),
        in_specs=[
            pl.BlockSpec((NC, _BN, DHID // NC), lambda i: (0, i, 0)),
            pl.BlockSpec((NC, _BN, DHID // NC), lambda i: (0, i, 0)),
            pl.BlockSpec((_BN, 2), lambda i: (i, 0)),
            pl.BlockSpec((1, DHID), lambda i: (0, 0)),
            pl.BlockSpec((DHID, DOUT), lambda i: (0, 0)),
        ],
        out_specs=pl.BlockSpec((_BN, DOUT), lambda i: (i, 0)),
        out_shape=jax.ShapeDtypeStruct((NPAD, DOUT), jnp.float32),
    )(agg1, h1p, deg_t, b1, w2)


def _tc3_body(agg_ref, hp_ref, deg_ref, b2_ref, out_ref):
    dis = _dis(deg_ref)
    s = agg_ref[0] + agg_ref[1] + hp_ref[...]
    out_ref[...] = jnp.maximum(dis * s + b2_ref[...], 0.0)


def _tc3(agg2, h2p, deg_t, b2):
    return pl.pallas_call(
        _tc3_body,
        grid=(NPAD // _BN,),
        in_specs=[
            pl.BlockSpec((NC, _BN, DOUT), lambda i: (0, i, 0)),
            pl.BlockSpec((_BN, DOUT), lambda i: (i, 0)),
            pl.BlockSpec((_BN, 2), lambda i: (i, 0)),
            pl.BlockSpec((1, DOUT), lambda i: (0, 0)),
        ],
        out_specs=pl.BlockSpec((_BN, DOUT), lambda i: (i, 0)),
        out_shape=jax.ShapeDtypeStruct((NPAD, DOUT), jnp.float32),
    )(agg2, h2p, deg_t, b2)


def kernel(x, edge_index, W1, b1, W2, b2):
    src = edge_index[0]
    dst = edge_index[1]

    pad = jnp.full((EPAD - E,), N, dtype=src.dtype)
    src2d = jnp.concatenate([src, pad]).reshape(EROWS, 128)
    dst2d = jnp.concatenate([dst, pad]).reshape(EROWS, 128)

    deg2 = _deg_partials(dst2d)
    deg_t = deg2.reshape(NC, NPAD, 128)[:, :, 0].T

    x_pad = jnp.zeros((NPAD, DIN), x.dtype).at[:N].set(x)

    h1p = _tc1(x_pad, W1, deg_t)
    agg1 = _agg128(h1p.reshape(NC * NPAD, DHID // NC), src2d, dst2d)
    agg1 = agg1.reshape(NC, NPAD, DHID // NC)
    h2p = _tc2(agg1, h1p, deg_t, b1.reshape(1, DHID), W2)
    agg2 = _agg_edge(h2p, src2d, dst2d).reshape(NC, NPAD, DOUT)
    out = _tc3(agg2, h2p, deg_t, b2.reshape(1, DOUT))
    return out[:N]


# speedup vs baseline: 7.8993x; 1.0372x over previous
import functools

import jax
import jax.numpy as jnp
from jax import lax
from jax.experimental import pallas as pl
from jax.experimental.pallas import tpu as pltpu
from jax.experimental.pallas import tpu_sc as plsc

N = 10000
NPAD = 10240
E = 320000
DIN = 128
DHID = 256
DOUT = 128

NC = 2
NS = 16
L = 16
NW = NC * NS

EROWS = 2560
EPAD = EROWS * 128
RPT = EROWS // NS
NROWS_PT = NPAD // NS


def _sc_mesh():
    return plsc.VectorSubcoreMesh(
        core_axis_name="c", subcore_axis_name="s", num_cores=NC, num_subcores=NS
    )


_RPW = EROWS // NW


def _deg_partials(dst2d):
    @functools.partial(
        pl.kernel,
        mesh=_sc_mesh(),
        out_type=jax.ShapeDtypeStruct((NC * NPAD, 128), jnp.float32),
        scratch_types=[
            pltpu.VMEM((_RPW, 128), jnp.int32),
            pltpu.VMEM((128, 128), jnp.float32),
            pltpu.VMEM_SHARED((NPAD, 128), jnp.float32),
        ],
    )
    def deg_kernel(dst_hbm, out_hbm, di_v, rows_v, deg_sh):
        cid = lax.axis_index("c")
        sid = lax.axis_index("s")
        rbase = (cid * NS + sid) * _RPW
        obase = sid * NROWS_PT

        def zrow(i, _):
            def zvec(k, _):
                rows_v[i, pl.ds(k * L, L)] = jnp.zeros((L,), jnp.float32)
                return 0

            lax.fori_loop(0, 128 // L, zvec, 0)
            return 0

        lax.fori_loop(0, 128, zrow, 0)

        def clr(k, _):
            pltpu.sync_copy(rows_v, deg_sh.at[pl.ds(obase + k * 128, 128)])
            return 0

        lax.fori_loop(0, NROWS_PT // 128, clr, 0)
        plsc.subcore_barrier()

        def set1(i, _):
            def svec(k, _):
                rows_v[i, pl.ds(k * L, L)] = jnp.ones((L,), jnp.float32)
                return 0

            lax.fori_loop(0, 128 // L, svec, 0)
            return 0

        lax.fori_loop(0, 128, set1, 0)

        pltpu.sync_copy(dst_hbm.at[pl.ds(rbase, _RPW)], di_v)

        def main(j, _):
            pltpu.sync_copy(rows_v, deg_sh.at[di_v.at[j]], add=True)
            return 0

        lax.fori_loop(0, _RPW, main, 0)
        plsc.subcore_barrier()

        pltpu.sync_copy(deg_sh.at[pl.ds(obase, NROWS_PT)],
                        out_hbm.at[pl.ds(cid * NPAD + obase, NROWS_PT)])

    return deg_kernel(dst2d)


CHR = 16


def _make_agg(dh):
    @functools.partial(
        pl.kernel,
        mesh=_sc_mesh(),
        out_type=jax.ShapeDtypeStruct((NC * NPAD, dh), jnp.float32),
        scratch_types=[
            pltpu.VMEM((CHR, 128), jnp.int32),
            pltpu.VMEM((CHR, 128), jnp.int32),
            pltpu.VMEM((128, dh), jnp.float32),
            pltpu.VMEM((128, dh), jnp.float32),
            pltpu.VMEM_SHARED((NPAD, dh), jnp.float32),
            pltpu.SemaphoreType.DMA,
            pltpu.SemaphoreType.DMA,
        ],
    )
    def agg_kernel(h_hbm, src_hbm, dst_hbm, out_hbm, si_v, di_v, rows_a, rows_b,
                   agg_sh, sem_a, sem_b):
        cid = lax.axis_index("c")
        sid = lax.axis_index("s")
        rbase = sid * RPT
        off = cid * NPAD

        def zrow(i, _):
            def zvec(k, _):
                rows_a[i, pl.ds(k * L, L)] = jnp.zeros((L,), jnp.float32)
                return 0

            lax.fori_loop(0, dh // L, zvec, 0)
            return 0

        lax.fori_loop(0, 128, zrow, 0)

        def clr(k, _):
            pltpu.sync_copy(rows_a, agg_sh.at[pl.ds(sid * NROWS_PT + k * 128, 128)])
            return 0

        lax.fori_loop(0, NROWS_PT // 128, clr, 0)
        plsc.subcore_barrier()

        def chunk(q, _):
            rq = rbase + q * CHR
            pltpu.sync_copy(src_hbm.at[pl.ds(rq, CHR)], si_v)
            pltpu.sync_copy(dst_hbm.at[pl.ds(rq, CHR)], di_v)

            def off_row(i, _):
                def off_vec(k, _):
                    sl = pl.ds(k * L, L)
                    si_v[i, sl] = si_v[i, sl] + off
                    return 0

                lax.fori_loop(0, 128 // L, off_vec, 0)
                return 0

            lax.fori_loop(0, CHR, off_row, 0)

            def pair(p, _):
                cpa = pltpu.async_copy(h_hbm.at[si_v.at[2 * p]], rows_a, sem_a)
                cpb = pltpu.async_copy(h_hbm.at[si_v.at[2 * p + 1]], rows_b, sem_b)
                cpa.wait()
                pltpu.sync_copy(rows_a, agg_sh.at[di_v.at[2 * p]], add=True)
                cpb.wait()
                pltpu.sync_copy(rows_b, agg_sh.at[di_v.at[2 * p + 1]], add=True)
                return 0

            lax.fori_loop(0, CHR // 2, pair, 0)
            return 0

        lax.fori_loop(0, RPT // CHR, chunk, 0)
        plsc.subcore_barrier()

        pltpu.sync_copy(
            agg_sh.at[pl.ds(sid * NROWS_PT, NROWS_PT)],
            out_hbm.at[pl.ds(cid * NPAD + sid * NROWS_PT, NROWS_PT)],
        )

    return agg_kernel


_agg128 = _make_agg(DHID // NC)


_RPT_ES = EROWS // NW


def _agg_edge(h, src2d, dst2d):
    @functools.partial(
        pl.kernel,
        mesh=_sc_mesh(),
        out_type=jax.ShapeDtypeStruct((NC * NPAD, DOUT), jnp.float32),
        scratch_types=[
            pltpu.VMEM((CHR, 128), jnp.int32),
            pltpu.VMEM((CHR, 128), jnp.int32),
            pltpu.VMEM((128, DOUT), jnp.float32),
            pltpu.VMEM((128, DOUT), jnp.float32),
            pltpu.VMEM_SHARED((NPAD, DOUT), jnp.float32),
            pltpu.SemaphoreType.DMA,
            pltpu.SemaphoreType.DMA,
        ],
    )
    def agg_kernel(h_hbm, src_hbm, dst_hbm, out_hbm, si_v, di_v, rows_a, rows_b,
                   agg_sh, sem_a, sem_b):
        cid = lax.axis_index("c")
        sid = lax.axis_index("s")
        rbase = (cid * NS + sid) * _RPT_ES

        def zrow(i, _):
            def zvec(k, _):
                rows_a[i, pl.ds(k * L, L)] = jnp.zeros((L,), jnp.float32)
                return 0

            lax.fori_loop(0, DOUT // L, zvec, 0)
            return 0

        lax.fori_loop(0, 128, zrow, 0)

        def clr(k, _):
            pltpu.sync_copy(rows_a, agg_sh.at[pl.ds(sid * NROWS_PT + k * 128, 128)])
            return 0

        lax.fori_loop(0, NROWS_PT // 128, clr, 0)
        plsc.subcore_barrier()

        def chunk(q, _):
            rq = rbase + q * CHR
            pltpu.sync_copy(src_hbm.at[pl.ds(rq, CHR)], si_v)
            pltpu.sync_copy(dst_hbm.at[pl.ds(rq, CHR)], di_v)

            def pair(p, _):
                cpa = pltpu.async_copy(h_hbm.at[si_v.at[2 * p]], rows_a, sem_a)
                cpb = pltpu.async_copy(h_hbm.at[si_v.at[2 * p + 1]], rows_b, sem_b)
                cpa.wait()
                pltpu.sync_copy(rows_a, agg_sh.at[di_v.at[2 * p]], add=True)
                cpb.wait()
                pltpu.sync_copy(rows_b, agg_sh.at[di_v.at[2 * p + 1]], add=True)
                return 0

            lax.fori_loop(0, CHR // 2, pair, 0)
            return 0

        lax.fori_loop(0, _RPT_ES // CHR, chunk, 0)
        plsc.subcore_barrier()

        pltpu.sync_copy(
            agg_sh.at[pl.ds(sid * NROWS_PT, NROWS_PT)],
            out_hbm.at[pl.ds(cid * NPAD + sid * NROWS_PT, NROWS_PT)],
        )

    return agg_kernel(h, src2d, dst2d)


_BN = 512


def _dis(deg_ref):
    return lax.rsqrt(deg_ref[:, 0:1] + deg_ref[:, 1:2] + 1.0)


def _tc1_body(x_ref, w_ref, deg_ref, out_ref):
    h = jnp.dot(x_ref[...], w_ref[...], preferred_element_type=jnp.float32)
    out_ref[0] = _dis(deg_ref) * h


def _tc1(x_pad, w1, deg_t):
    return pl.pallas_call(
        _tc1_body,
        grid=(NC, NPAD // _BN),
        in_specs=[
            pl.BlockSpec((_BN, DIN), lambda c, i: (i, 0)),
            pl.BlockSpec((DIN, DHID // NC), lambda c, i: (0, c)),
            pl.BlockSpec((_BN, 2), lambda c, i: (i, 0)),
        ],
        out_specs=pl.BlockSpec((1, _BN, DHID // NC), lambda c, i: (c, i, 0)),
        out_shape=jax.ShapeDtypeStruct((NC, NPAD, DHID // NC), jnp.float32),
    )(x_pad, w1, deg_t)


def _tc2_body(agg_ref, hp_ref, deg_ref, b1_ref, w2_ref, out_ref):
    dis = _dis(deg_ref)
    b = b1_ref[...]
    za = jnp.maximum(dis * (agg_ref[0] + hp_ref[0]) + b[:, :128], 0.0)
    zb = jnp.maximum(dis * (agg_ref[1] + hp_ref[1]) + b[:, 128:], 0.0)
    w = w2_ref[...]
    h2 = jnp.dot(za, w[:128], preferred_element_type=jnp.float32)
    h2 = h2 + jnp.dot(zb, w[128:], preferred_element_type=jnp.float32)
    out_ref[...] = dis * h2


def _tc2(agg1, h1p, deg_t, b1, w2):
    return pl.pallas_call(
        _tc2_body,
        grid=(NPAD // _BN,),
        in_specs=[
            pl.BlockSpec((NC, _BN, DHID // NC), lambda i: (0, i, 0)),
            pl.BlockSpec((NC, _BN, DHID // NC), lambda i: (0, i, 0)),
            pl.BlockSpec((_BN, 2), lambda i: (i, 0)),
            pl.BlockSpec((1, DHID), lambda i: (0, 0)),
            pl.BlockSpec((DHID, DOUT), lambda i: (0, 0)),
        ],
        out_specs=pl.BlockSpec((_BN, DOUT), lambda i: (i, 0)),
        out_shape=jax.ShapeDtypeStruct((NPAD, DOUT), jnp.float32),
    )(agg1, h1p, deg_t, b1, w2)


def _tc3_body(agg_ref, hp_ref, deg_ref, b2_ref, out_ref):
    dis = _dis(deg_ref)
    s = agg_ref[0] + agg_ref[1] + hp_ref[...]
    out_ref[...] = jnp.maximum(dis * s + b2_ref[...], 0.0)


def _tc3(agg2, h2p, deg_t, b2):
    return pl.pallas_call(
        _tc3_body,
        grid=(NPAD // _BN,),
        in_specs=[
            pl.BlockSpec((NC, _BN, DOUT), lambda i: (0, i, 0)),
            pl.BlockSpec((_BN, DOUT), lambda i: (i, 0)),
            pl.BlockSpec((_BN, 2), lambda i: (i, 0)),
            pl.BlockSpec((1, DOUT), lambda i: (0, 0)),
        ],
        out_specs=pl.BlockSpec((_BN, DOUT), lambda i: (i, 0)),
        out_shape=jax.ShapeDtypeStruct((NPAD, DOUT), jnp.float32),
    )(agg2, h2p, deg_t, b2)


def kernel(x, edge_index, W1, b1, W2, b2):
    src = edge_index[0]
    dst = edge_index[1]

    pad = jnp.full((EPAD - E,), N, dtype=src.dtype)
    src2d = jnp.concatenate([src, pad]).reshape(EROWS, 128)
    dst2d = jnp.concatenate([dst, pad]).reshape(EROWS, 128)

    deg2 = _deg_partials(dst2d)
    deg_t = deg2.reshape(NC, NPAD, 128)[:, :, 0].T

    x_pad = jnp.zeros((NPAD, DIN), x.dtype).at[:N].set(x)

    h1p = _tc1(x_pad, W1, deg_t)
    agg1 = _agg128(h1p.reshape(NC * NPAD, DHID // NC), src2d, dst2d)
    agg1 = agg1.reshape(NC, NPAD, DHID // NC)
    h2p = _tc2(agg1, h1p, deg_t, b1.reshape(1, DHID), W2)
    agg2 = _agg_edge(h2p, src2d, dst2d).reshape(NC, NPAD, DOUT)
    out = _tc3(agg2, h2p, deg_t, b2.reshape(1, DOUT))
    return out[:N]


# speedup vs baseline: 16.6213x; 2.1041x over previous
import functools

import jax
import jax.numpy as jnp
from jax import lax
from jax.experimental import pallas as pl
from jax.experimental.pallas import tpu as pltpu
from jax.experimental.pallas import tpu_sc as plsc

N = 10000
NPAD = 10240
E = 320000
DIN = 128
DHID = 256
DOUT = 128

NC = 2
NS = 16
L = 16
NW = NC * NS

EROWS = 2560
EPAD = EROWS * 128
RPT = EROWS // NS
NROWS_PT = NPAD // NS


def _sc_mesh():
    return plsc.VectorSubcoreMesh(
        core_axis_name="c", subcore_axis_name="s", num_cores=NC, num_subcores=NS
    )


_RPW = EROWS // NW


def _deg_partials(dst2d):
    @functools.partial(
        pl.kernel,
        mesh=_sc_mesh(),
        out_type=jax.ShapeDtypeStruct((NC * NPAD, 128), jnp.float32),
        scratch_types=[
            pltpu.VMEM((_RPW, 128), jnp.int32),
            pltpu.VMEM((128, 128), jnp.float32),
            pltpu.VMEM_SHARED((NPAD, 128), jnp.float32),
        ],
    )
    def deg_kernel(dst_hbm, out_hbm, di_v, rows_v, deg_sh):
        cid = lax.axis_index("c")
        sid = lax.axis_index("s")
        rbase = (cid * NS + sid) * _RPW
        obase = sid * NROWS_PT

        def zrow(i, _):
            def zvec(k, _):
                rows_v[i, pl.ds(k * L, L)] = jnp.zeros((L,), jnp.float32)
                return 0

            lax.fori_loop(0, 128 // L, zvec, 0)
            return 0

        lax.fori_loop(0, 128, zrow, 0)

        def clr(k, _):
            pltpu.sync_copy(rows_v, deg_sh.at[pl.ds(obase + k * 128, 128)])
            return 0

        lax.fori_loop(0, NROWS_PT // 128, clr, 0)
        plsc.subcore_barrier()

        def set1(i, _):
            def svec(k, _):
                rows_v[i, pl.ds(k * L, L)] = jnp.ones((L,), jnp.float32)
                return 0

            lax.fori_loop(0, 128 // L, svec, 0)
            return 0

        lax.fori_loop(0, 128, set1, 0)

        pltpu.sync_copy(dst_hbm.at[pl.ds(rbase, _RPW)], di_v)

        def main(j, _):
            pltpu.sync_copy(rows_v, deg_sh.at[di_v.at[j]], add=True)
            return 0

        lax.fori_loop(0, _RPW, main, 0)
        plsc.subcore_barrier()

        pltpu.sync_copy(deg_sh.at[pl.ds(obase, NROWS_PT)],
                        out_hbm.at[pl.ds(cid * NPAD + obase, NROWS_PT)])

    return deg_kernel(dst2d)


CHR = 16


def _make_agg(dh):
    @functools.partial(
        pl.kernel,
        mesh=_sc_mesh(),
        out_type=jax.ShapeDtypeStruct((NC * NPAD, dh), jnp.float32),
        scratch_types=[
            pltpu.VMEM((CHR, 128), jnp.int32),
            pltpu.VMEM((CHR, 128), jnp.int32),
            pltpu.VMEM((128, dh), jnp.float32),
            pltpu.VMEM((128, dh), jnp.float32),
            pltpu.VMEM_SHARED((NPAD, dh), jnp.float32),
            pltpu.SemaphoreType.DMA,
            pltpu.SemaphoreType.DMA,
        ],
    )
    def agg_kernel(h_hbm, src_hbm, dst_hbm, out_hbm, si_v, di_v, rows_a, rows_b,
                   agg_sh, sem_a, sem_b):
        cid = lax.axis_index("c")
        sid = lax.axis_index("s")
        rbase = sid * RPT
        off = cid * NPAD

        def zrow(i, _):
            def zvec(k, _):
                rows_a[i, pl.ds(k * L, L)] = jnp.zeros((L,), jnp.float32)
                return 0

            lax.fori_loop(0, dh // L, zvec, 0)
            return 0

        lax.fori_loop(0, 128, zrow, 0)

        def clr(k, _):
            pltpu.sync_copy(rows_a, agg_sh.at[pl.ds(sid * NROWS_PT + k * 128, 128)])
            return 0

        lax.fori_loop(0, NROWS_PT // 128, clr, 0)
        plsc.subcore_barrier()

        def chunk(q, _):
            rq = rbase + q * CHR
            pltpu.sync_copy(src_hbm.at[pl.ds(rq, CHR)], si_v)
            pltpu.sync_copy(dst_hbm.at[pl.ds(rq, CHR)], di_v)

            def off_row(i, _):
                def off_vec(k, _):
                    sl = pl.ds(k * L, L)
                    si_v[i, sl] = si_v[i, sl] + off
                    return 0

                lax.fori_loop(0, 128 // L, off_vec, 0)
                return 0

            lax.fori_loop(0, CHR, off_row, 0)

            def pair(p, _):
                cpa = pltpu.async_copy(h_hbm.at[si_v.at[2 * p]], rows_a, sem_a)
                cpb = pltpu.async_copy(h_hbm.at[si_v.at[2 * p + 1]], rows_b, sem_b)
                cpa.wait()
                pltpu.sync_copy(rows_a, agg_sh.at[di_v.at[2 * p]], add=True)
                cpb.wait()
                pltpu.sync_copy(rows_b, agg_sh.at[di_v.at[2 * p + 1]], add=True)
                return 0

            lax.fori_loop(0, CHR // 2, pair, 0)
            return 0

        lax.fori_loop(0, RPT // CHR, chunk, 0)
        plsc.subcore_barrier()

        pltpu.sync_copy(
            agg_sh.at[pl.ds(sid * NROWS_PT, NROWS_PT)],
            out_hbm.at[pl.ds(cid * NPAD + sid * NROWS_PT, NROWS_PT)],
        )

    return agg_kernel


_agg128 = _make_agg(DHID // NC)


_RPT_ES = EROWS // NW


def _agg_edge(h, src2d, dst2d):
    @functools.partial(
        pl.kernel,
        mesh=_sc_mesh(),
        out_type=jax.ShapeDtypeStruct((NC * NPAD, DOUT), jnp.float32),
        scratch_types=[
            pltpu.VMEM((CHR, 128), jnp.int32),
            pltpu.VMEM((CHR, 128), jnp.int32),
            pltpu.VMEM((128, DOUT), jnp.float32),
            pltpu.VMEM((128, DOUT), jnp.float32),
            pltpu.VMEM_SHARED((NPAD, DOUT), jnp.float32),
            pltpu.SemaphoreType.DMA,
            pltpu.SemaphoreType.DMA,
        ],
    )
    def agg_kernel(h_hbm, src_hbm, dst_hbm, out_hbm, si_v, di_v, rows_a, rows_b,
                   agg_sh, sem_a, sem_b):
        cid = lax.axis_index("c")
        sid = lax.axis_index("s")
        rbase = (cid * NS + sid) * _RPT_ES

        def zrow(i, _):
            def zvec(k, _):
                rows_a[i, pl.ds(k * L, L)] = jnp.zeros((L,), jnp.float32)
                return 0

            lax.fori_loop(0, DOUT // L, zvec, 0)
            return 0

        lax.fori_loop(0, 128, zrow, 0)

        def clr(k, _):
            pltpu.sync_copy(rows_a, agg_sh.at[pl.ds(sid * NROWS_PT + k * 128, 128)])
            return 0

        lax.fori_loop(0, NROWS_PT // 128, clr, 0)
        plsc.subcore_barrier()

        def chunk(q, _):
            rq = rbase + q * CHR
            pltpu.sync_copy(src_hbm.at[pl.ds(rq, CHR)], si_v)
            pltpu.sync_copy(dst_hbm.at[pl.ds(rq, CHR)], di_v)

            def pair(p, _):
                cpa = pltpu.async_copy(h_hbm.at[si_v.at[2 * p]], rows_a, sem_a)
                cpb = pltpu.async_copy(h_hbm.at[si_v.at[2 * p + 1]], rows_b, sem_b)
                cpa.wait()
                pltpu.sync_copy(rows_a, agg_sh.at[di_v.at[2 * p]], add=True)
                cpb.wait()
                pltpu.sync_copy(rows_b, agg_sh.at[di_v.at[2 * p + 1]], add=True)
                return 0

            lax.fori_loop(0, CHR // 2, pair, 0)
            return 0

        lax.fori_loop(0, _RPT_ES // CHR, chunk, 0)
        plsc.subcore_barrier()

        pltpu.sync_copy(
            agg_sh.at[pl.ds(sid * NROWS_PT, NROWS_PT)],
            out_hbm.at[pl.ds(cid * NPAD + sid * NROWS_PT, NROWS_PT)],
        )

    return agg_kernel(h, src2d, dst2d)


_BN = 512


def _dis(deg_ref):
    return lax.rsqrt(deg_ref[:, 0:1] + deg_ref[:, 1:2] + 1.0)


def _tc1_body(x_ref, w_ref, deg_ref, out_ref):
    h = jnp.dot(x_ref[...], w_ref[...], preferred_element_type=jnp.float32)
    out_ref[0] = _dis(deg_ref) * h


def _tc1(x_pad, w1, deg_t):
    return pl.pallas_call(
        _tc1_body,
        grid=(NC, NPAD // _BN),
        in_specs=[
            pl.BlockSpec((_BN, DIN), lambda c, i: (i, 0)),
            pl.BlockSpec((DIN, DHID // NC), lambda c, i: (0, c)),
            pl.BlockSpec((_BN, 2), lambda c, i: (i, 0)),
        ],
        out_specs=pl.BlockSpec((1, _BN, DHID // NC), lambda c, i: (c, i, 0)),
        out_shape=jax.ShapeDtypeStruct((NC, NPAD, DHID // NC), jnp.float32),
    )(x_pad, w1, deg_t)


def _tc2_body(agg_ref, hp_ref, deg_ref, b1_ref, w2_ref, out_ref):
    dis = _dis(deg_ref)
    b = b1_ref[...]
    za = jnp.maximum(dis * (agg_ref[0] + hp_ref[0]) + b[:, :128], 0.0)
    zb = jnp.maximum(dis * (agg_ref[1] + hp_ref[1]) + b[:, 128:], 0.0)
    w = w2_ref[...]
    h2 = jnp.dot(za, w[:128], preferred_element_type=jnp.float32)
    h2 = h2 + jnp.dot(zb, w[128:], preferred_element_type=jnp.float32)
    out_ref[...] = dis * h2


def _tc2(agg1, h1p, deg_t, b1, w2):
    return pl.pallas_call(
        _tc2_body,
        grid=(NPAD // _BN,),
        in_specs=[
            pl.BlockSpec((NC, _BN, DHID // NC), lambda i: (0, i, 0)),
            pl.BlockSpec((NC, _BN, DHID // NC), lambda i: (0, i, 0)),
            pl.BlockSpec((_BN, 2), lambda i: (i, 0)),
            pl.BlockSpec((1, DHID), lambda i: (0, 0)),
            pl.BlockSpec((DHID, DOUT), lambda i: (0, 0)),
        ],
        out_specs=pl.BlockSpec((_BN, DOUT), lambda i: (i, 0)),
        out_shape=jax.ShapeDtypeStruct((NPAD, DOUT), jnp.float32),
    )(agg1, h1p, deg_t, b1, w2)


def _tc3_body(agg_ref, hp_ref, deg_ref, b2_ref, out_ref):
    dis = _dis(deg_ref)
    s = agg_ref[0] + agg_ref[1] + hp_ref[...]
    out_ref[...] = jnp.maximum(dis * s + b2_ref[...], 0.0)


def _tc3(agg2, h2p, deg_t, b2):
    return pl.pallas_call(
        _tc3_body,
        grid=(NPAD // _BN,),
        in_specs=[
            pl.BlockSpec((NC, _BN, DOUT), lambda i: (0, i, 0)),
            pl.BlockSpec((_BN, DOUT), lambda i: (i, 0)),
            pl.BlockSpec((_BN, 2), lambda i: (i, 0)),
            pl.BlockSpec((1, DOUT), lambda i: (0, 0)),
        ],
        out_specs=pl.BlockSpec((_BN, DOUT), lambda i: (i, 0)),
        out_shape=jax.ShapeDtypeStruct((NPAD, DOUT), jnp.float32),
    )(agg2, h2p, deg_t, b2)


def kernel(x, edge_index, W1, b1, W2, b2):
    src = edge_index[0]
    dst = edge_index[1]

    pad = N + (jnp.arange(EPAD - E, dtype=src.dtype) % (NPAD - N))
    src2d = jnp.concatenate([src, pad]).reshape(EROWS, 128)
    dst2d = jnp.concatenate([dst, pad]).reshape(EROWS, 128)

    deg2 = _deg_partials(dst2d)
    deg_t = deg2.reshape(NC, NPAD, 128)[:, :, 0].T

    x_pad = jnp.zeros((NPAD, DIN), x.dtype).at[:N].set(x)

    h1p = _tc1(x_pad, W1, deg_t)
    agg1 = _agg128(h1p.reshape(NC * NPAD, DHID // NC), src2d, dst2d)
    agg1 = agg1.reshape(NC, NPAD, DHID // NC)
    h2p = _tc2(agg1, h1p, deg_t, b1.reshape(1, DHID), W2)
    agg2 = _agg_edge(h2p, src2d, dst2d).reshape(NC, NPAD, DOUT)
    out = _tc3(agg2, h2p, deg_t, b2.reshape(1, DOUT))
    return out[:N]


# speedup vs baseline: 21.5088x; 1.2941x over previous
import functools

import jax
import jax.numpy as jnp
from jax import lax
from jax.experimental import pallas as pl
from jax.experimental.pallas import tpu as pltpu
from jax.experimental.pallas import tpu_sc as plsc

N = 10000
NPAD = 10240
E = 320000
DIN = 128
DHID = 256
DOUT = 128

NC = 2
NS = 16
L = 16
NW = NC * NS

EROWS = 2560
EPAD = EROWS * 128
RPT = EROWS // NS
NROWS_PT = NPAD // NS


def _sc_mesh():
    return plsc.VectorSubcoreMesh(
        core_axis_name="c", subcore_axis_name="s", num_cores=NC, num_subcores=NS
    )


_RPW = EROWS // NW


def _deg_partials(dst2d):
    @functools.partial(
        pl.kernel,
        mesh=_sc_mesh(),
        out_type=jax.ShapeDtypeStruct((NC * NPAD, 128), jnp.float32),
        scratch_types=[
            pltpu.VMEM((_RPW, 128), jnp.int32),
            pltpu.VMEM((128, 128), jnp.float32),
            pltpu.VMEM_SHARED((NPAD, 128), jnp.float32),
        ],
    )
    def deg_kernel(dst_hbm, out_hbm, di_v, rows_v, deg_sh):
        cid = lax.axis_index("c")
        sid = lax.axis_index("s")
        rbase = (cid * NS + sid) * _RPW
        obase = sid * NROWS_PT

        def zrow(i, _):
            def zvec(k, _):
                rows_v[i, pl.ds(k * L, L)] = jnp.zeros((L,), jnp.float32)
                return 0

            lax.fori_loop(0, 128 // L, zvec, 0)
            return 0

        lax.fori_loop(0, 128, zrow, 0)

        def clr(k, _):
            pltpu.sync_copy(rows_v, deg_sh.at[pl.ds(obase + k * 128, 128)])
            return 0

        lax.fori_loop(0, NROWS_PT // 128, clr, 0)
        plsc.subcore_barrier()

        def set1(i, _):
            def svec(k, _):
                rows_v[i, pl.ds(k * L, L)] = jnp.ones((L,), jnp.float32)
                return 0

            lax.fori_loop(0, 128 // L, svec, 0)
            return 0

        lax.fori_loop(0, 128, set1, 0)

        pltpu.sync_copy(dst_hbm.at[pl.ds(rbase, _RPW)], di_v)

        def main(j, _):
            pltpu.sync_copy(rows_v, deg_sh.at[di_v.at[j]], add=True)
            return 0

        lax.fori_loop(0, _RPW, main, 0)
        plsc.subcore_barrier()

        pltpu.sync_copy(deg_sh.at[pl.ds(obase, NROWS_PT)],
                        out_hbm.at[pl.ds(cid * NPAD + obase, NROWS_PT)])

    return deg_kernel(dst2d)


CHR = 16


_RPT_ES = EROWS // NW


def _agg_edge(h, src2d, dst2d):
    @functools.partial(
        pl.kernel,
        mesh=_sc_mesh(),
        out_type=jax.ShapeDtypeStruct((NC * NPAD, DOUT), jnp.float32),
        scratch_types=[
            pltpu.VMEM((CHR, 128), jnp.int32),
            pltpu.VMEM((CHR, 128), jnp.int32),
            pltpu.VMEM((128, DOUT), jnp.float32),
            pltpu.VMEM((128, DOUT), jnp.float32),
            pltpu.VMEM_SHARED((NPAD, DOUT), jnp.float32),
            pltpu.SemaphoreType.DMA,
            pltpu.SemaphoreType.DMA,
        ],
    )
    def agg_kernel(h_hbm, src_hbm, dst_hbm, out_hbm, si_v, di_v, rows_a, rows_b,
                   agg_sh, sem_a, sem_b):
        cid = lax.axis_index("c")
        sid = lax.axis_index("s")
        rbase = (cid * NS + sid) * _RPT_ES

        def zrow(i, _):
            def zvec(k, _):
                rows_a[i, pl.ds(k * L, L)] = jnp.zeros((L,), jnp.float32)
                return 0

            lax.fori_loop(0, DOUT // L, zvec, 0)
            return 0

        lax.fori_loop(0, 128, zrow, 0)

        def clr(k, _):
            pltpu.sync_copy(rows_a, agg_sh.at[pl.ds(sid * NROWS_PT + k * 128, 128)])
            return 0

        lax.fori_loop(0, NROWS_PT // 128, clr, 0)
        plsc.subcore_barrier()

        def chunk(q, _):
            rq = rbase + q * CHR
            pltpu.sync_copy(src_hbm.at[pl.ds(rq, CHR)], si_v)
            pltpu.sync_copy(dst_hbm.at[pl.ds(rq, CHR)], di_v)

            def pair(p, _):
                cpa = pltpu.async_copy(h_hbm.at[si_v.at[2 * p]], rows_a, sem_a)
                cpb = pltpu.async_copy(h_hbm.at[si_v.at[2 * p + 1]], rows_b, sem_b)
                cpa.wait()
                pltpu.sync_copy(rows_a, agg_sh.at[di_v.at[2 * p]], add=True)
                cpb.wait()
                pltpu.sync_copy(rows_b, agg_sh.at[di_v.at[2 * p + 1]], add=True)
                return 0

            lax.fori_loop(0, CHR // 2, pair, 0)
            return 0

        lax.fori_loop(0, _RPT_ES // CHR, chunk, 0)
        plsc.subcore_barrier()

        pltpu.sync_copy(
            agg_sh.at[pl.ds(sid * NROWS_PT, NROWS_PT)],
            out_hbm.at[pl.ds(cid * NPAD + sid * NROWS_PT, NROWS_PT)],
        )

    return agg_kernel(h, src2d, dst2d)


_BN = 512


def _dis(deg_ref):
    return lax.rsqrt(deg_ref[:, 0:1] + deg_ref[:, 1:2] + 1.0)


def _tca_body(x_ref, deg_ref, out_ref):
    out_ref[...] = _dis(deg_ref) * x_ref[...]


def _tca(x_pad, deg_t):
    return pl.pallas_call(
        _tca_body,
        grid=(NPAD // _BN,),
        in_specs=[
            pl.BlockSpec((_BN, DIN), lambda i: (i, 0)),
            pl.BlockSpec((_BN, 2), lambda i: (i, 0)),
        ],
        out_specs=pl.BlockSpec((_BN, DIN), lambda i: (i, 0)),
        out_shape=jax.ShapeDtypeStruct((NPAD, DIN), jnp.float32),
    )(x_pad, deg_t)


def _tcb_body(agg_ref, xp_ref, deg_ref, b1_ref, w1_ref, w2_ref, out_ref):
    dis = _dis(deg_ref)
    u = agg_ref[0] + agg_ref[1] + xp_ref[...]
    t = jnp.dot(u, w1_ref[...], preferred_element_type=jnp.float32)
    z = jnp.maximum(dis * t + b1_ref[...], 0.0)
    h2 = jnp.dot(z, w2_ref[...], preferred_element_type=jnp.float32)
    out_ref[...] = dis * h2


def _tcb(agg1, xp, deg_t, b1, w1, w2):
    return pl.pallas_call(
        _tcb_body,
        grid=(NPAD // _BN,),
        in_specs=[
            pl.BlockSpec((NC, _BN, DIN), lambda i: (0, i, 0)),
            pl.BlockSpec((_BN, DIN), lambda i: (i, 0)),
            pl.BlockSpec((_BN, 2), lambda i: (i, 0)),
            pl.BlockSpec((1, DHID), lambda i: (0, 0)),
            pl.BlockSpec((DIN, DHID), lambda i: (0, 0)),
            pl.BlockSpec((DHID, DOUT), lambda i: (0, 0)),
        ],
        out_specs=pl.BlockSpec((_BN, DOUT), lambda i: (i, 0)),
        out_shape=jax.ShapeDtypeStruct((NPAD, DOUT), jnp.float32),
    )(agg1, xp, deg_t, b1, w1, w2)


def _tc3_body(agg_ref, hp_ref, deg_ref, b2_ref, out_ref):
    dis = _dis(deg_ref)
    s = agg_ref[0] + agg_ref[1] + hp_ref[...]
    out_ref[...] = jnp.maximum(dis * s + b2_ref[...], 0.0)


def _tc3(agg2, h2p, deg_t, b2):
    return pl.pallas_call(
        _tc3_body,
        grid=(NPAD // _BN,),
        in_specs=[
            pl.BlockSpec((NC, _BN, DOUT), lambda i: (0, i, 0)),
            pl.BlockSpec((_BN, DOUT), lambda i: (i, 0)),
            pl.BlockSpec((_BN, 2), lambda i: (i, 0)),
            pl.BlockSpec((1, DOUT), lambda i: (0, 0)),
        ],
        out_specs=pl.BlockSpec((_BN, DOUT), lambda i: (i, 0)),
        out_shape=jax.ShapeDtypeStruct((NPAD, DOUT), jnp.float32),
    )(agg2, h2p, deg_t, b2)


def kernel(x, edge_index, W1, b1, W2, b2):
    src = edge_index[0]
    dst = edge_index[1]

    pad = N + (jnp.arange(EPAD - E, dtype=src.dtype) % (NPAD - N))
    src2d = jnp.concatenate([src, pad]).reshape(EROWS, 128)
    dst2d = jnp.concatenate([dst, pad]).reshape(EROWS, 128)

    deg2 = _deg_partials(dst2d)
    deg_t = deg2.reshape(NC, NPAD, 128)[:, :, 0].T

    x_pad = jnp.zeros((NPAD, DIN), x.dtype).at[:N].set(x)

    xp = _tca(x_pad, deg_t)
    agg1 = _agg_edge(xp, src2d, dst2d).reshape(NC, NPAD, DIN)
    h2p = _tcb(agg1, xp, deg_t, b1.reshape(1, DHID), W1, W2)
    agg2 = _agg_edge(h2p, src2d, dst2d).reshape(NC, NPAD, DOUT)
    out = _tc3(agg2, h2p, deg_t, b2.reshape(1, DOUT))
    return out[:N]


# speedup vs baseline: 21.7155x; 1.0096x over previous
import functools

import jax
import jax.numpy as jnp
from jax import lax
from jax.experimental import pallas as pl
from jax.experimental.pallas import tpu as pltpu
from jax.experimental.pallas import tpu_sc as plsc

N = 10000
NPAD = 10240
E = 320000
DIN = 128
DHID = 256
DOUT = 128

NC = 2
NS = 16
L = 16
NW = NC * NS

EROWS = 2560
EPAD = EROWS * 128
RPT = EROWS // NS
NROWS_PT = NPAD // NS


def _sc_mesh():
    return plsc.VectorSubcoreMesh(
        core_axis_name="c", subcore_axis_name="s", num_cores=NC, num_subcores=NS
    )


_RPW = EROWS // NW


def _deg_partials(dst2d):
    @functools.partial(
        pl.kernel,
        mesh=_sc_mesh(),
        out_type=jax.ShapeDtypeStruct((NC * NPAD, 128), jnp.float32),
        scratch_types=[
            pltpu.VMEM((_RPW, 128), jnp.int32),
            pltpu.VMEM((128, 128), jnp.float32),
            pltpu.VMEM_SHARED((NPAD, 128), jnp.float32),
        ],
    )
    def deg_kernel(dst_hbm, out_hbm, di_v, rows_v, deg_sh):
        cid = lax.axis_index("c")
        sid = lax.axis_index("s")
        rbase = (cid * NS + sid) * _RPW
        obase = sid * NROWS_PT

        def zrow(i, _):
            def zvec(k, _):
                rows_v[i, pl.ds(k * L, L)] = jnp.zeros((L,), jnp.float32)
                return 0

            lax.fori_loop(0, 128 // L, zvec, 0)
            return 0

        lax.fori_loop(0, 128, zrow, 0)

        def clr(k, _):
            pltpu.sync_copy(rows_v, deg_sh.at[pl.ds(obase + k * 128, 128)])
            return 0

        lax.fori_loop(0, NROWS_PT // 128, clr, 0)
        plsc.subcore_barrier()

        def set1(i, _):
            def svec(k, _):
                rows_v[i, pl.ds(k * L, L)] = jnp.ones((L,), jnp.float32)
                return 0

            lax.fori_loop(0, 128 // L, svec, 0)
            return 0

        lax.fori_loop(0, 128, set1, 0)

        pltpu.sync_copy(dst_hbm.at[pl.ds(rbase, _RPW)], di_v)

        def main(j, _):
            pltpu.sync_copy(rows_v, deg_sh.at[di_v.at[j]], add=True)
            return 0

        lax.fori_loop(0, _RPW, main, 0)
        plsc.subcore_barrier()

        pltpu.sync_copy(deg_sh.at[pl.ds(obase, NROWS_PT)],
                        out_hbm.at[pl.ds(cid * NPAD + obase, NROWS_PT)])

    return deg_kernel(dst2d)


CHR = 16


_RPT_ES = EROWS // NW


def _agg_edge(h, src2d, dst2d):
    @functools.partial(
        pl.kernel,
        mesh=_sc_mesh(),
        out_type=jax.ShapeDtypeStruct((NC * NPAD, DOUT), jnp.float32),
        scratch_types=[
            pltpu.VMEM((CHR, 128), jnp.int32),
            pltpu.VMEM((CHR, 128), jnp.int32),
            pltpu.VMEM((128, DOUT), jnp.float32),
            pltpu.VMEM((128, DOUT), jnp.float32),
            pltpu.VMEM_SHARED((NPAD, DOUT), jnp.float32),
            pltpu.SemaphoreType.DMA,
            pltpu.SemaphoreType.DMA,
            pltpu.SemaphoreType.DMA,
            pltpu.SemaphoreType.DMA,
        ],
    )
    def agg_kernel(h_hbm, src_hbm, dst_hbm, out_hbm, si_v, di_v, rows_a, rows_b,
                   agg_sh, sem_a, sem_b, sem_sa, sem_sb):
        cid = lax.axis_index("c")
        sid = lax.axis_index("s")
        rbase = (cid * NS + sid) * _RPT_ES

        def zrow(i, _):
            def zvec(k, _):
                rows_a[i, pl.ds(k * L, L)] = jnp.zeros((L,), jnp.float32)
                return 0

            lax.fori_loop(0, DOUT // L, zvec, 0)
            return 0

        lax.fori_loop(0, 128, zrow, 0)

        def clr(k, _):
            pltpu.sync_copy(rows_a, agg_sh.at[pl.ds(sid * NROWS_PT + k * 128, 128)])
            return 0

        lax.fori_loop(0, NROWS_PT // 128, clr, 0)
        plsc.subcore_barrier()

        def chunk(q, _):
            rq = rbase + q * CHR
            pltpu.sync_copy(src_hbm.at[pl.ds(rq, CHR)], si_v)
            pltpu.sync_copy(dst_hbm.at[pl.ds(rq, CHR)], di_v)

            def pair(p, _):
                cpa = pltpu.async_copy(h_hbm.at[si_v.at[2 * p]], rows_a, sem_a)
                cpb = pltpu.async_copy(h_hbm.at[si_v.at[2 * p + 1]], rows_b, sem_b)
                cpa.wait()
                sca = pltpu.async_copy(rows_a, agg_sh.at[di_v.at[2 * p]],
                                       sem_sa, add=True)
                cpb.wait()
                scb = pltpu.async_copy(rows_b, agg_sh.at[di_v.at[2 * p + 1]],
                                       sem_sb, add=True)
                sca.wait()
                scb.wait()
                return 0

            lax.fori_loop(0, CHR // 2, pair, 0)
            return 0

        lax.fori_loop(0, _RPT_ES // CHR, chunk, 0)
        plsc.subcore_barrier()

        pltpu.sync_copy(
            agg_sh.at[pl.ds(sid * NROWS_PT, NROWS_PT)],
            out_hbm.at[pl.ds(cid * NPAD + sid * NROWS_PT, NROWS_PT)],
        )

    return agg_kernel(h, src2d, dst2d)


_BN = 512


def _dis(deg_ref):
    return lax.rsqrt(deg_ref[:, 0:1] + deg_ref[:, 1:2] + 1.0)


def _tca_body(x_ref, deg_ref, out_ref):
    out_ref[...] = _dis(deg_ref) * x_ref[...]


def _tca(x_pad, deg_t):
    return pl.pallas_call(
        _tca_body,
        grid=(NPAD // _BN,),
        in_specs=[
            pl.BlockSpec((_BN, DIN), lambda i: (i, 0)),
            pl.BlockSpec((_BN, 2), lambda i: (i, 0)),
        ],
        out_specs=pl.BlockSpec((_BN, DIN), lambda i: (i, 0)),
        out_shape=jax.ShapeDtypeStruct((NPAD, DIN), jnp.float32),
    )(x_pad, deg_t)


def _tcb_body(agg_ref, xp_ref, deg_ref, b1_ref, w1_ref, w2_ref, out_ref):
    dis = _dis(deg_ref)
    u = agg_ref[0] + agg_ref[1] + xp_ref[...]
    t = jnp.dot(u, w1_ref[...], preferred_element_type=jnp.float32)
    z = jnp.maximum(dis * t + b1_ref[...], 0.0)
    h2 = jnp.dot(z, w2_ref[...], preferred_element_type=jnp.float32)
    out_ref[...] = dis * h2


def _tcb(agg1, xp, deg_t, b1, w1, w2):
    return pl.pallas_call(
        _tcb_body,
        grid=(NPAD // _BN,),
        in_specs=[
            pl.BlockSpec((NC, _BN, DIN), lambda i: (0, i, 0)),
            pl.BlockSpec((_BN, DIN), lambda i: (i, 0)),
            pl.BlockSpec((_BN, 2), lambda i: (i, 0)),
            pl.BlockSpec((1, DHID), lambda i: (0, 0)),
            pl.BlockSpec((DIN, DHID), lambda i: (0, 0)),
            pl.BlockSpec((DHID, DOUT), lambda i: (0, 0)),
        ],
        out_specs=pl.BlockSpec((_BN, DOUT), lambda i: (i, 0)),
        out_shape=jax.ShapeDtypeStruct((NPAD, DOUT), jnp.float32),
    )(agg1, xp, deg_t, b1, w1, w2)


def _tc3_body(agg_ref, hp_ref, deg_ref, b2_ref, out_ref):
    dis = _dis(deg_ref)
    s = agg_ref[0] + agg_ref[1] + hp_ref[...]
    out_ref[...] = jnp.maximum(dis * s + b2_ref[...], 0.0)


def _tc3(agg2, h2p, deg_t, b2):
    return pl.pallas_call(
        _tc3_body,
        grid=(NPAD // _BN,),
        in_specs=[
            pl.BlockSpec((NC, _BN, DOUT), lambda i: (0, i, 0)),
            pl.BlockSpec((_BN, DOUT), lambda i: (i, 0)),
            pl.BlockSpec((_BN, 2), lambda i: (i, 0)),
            pl.BlockSpec((1, DOUT), lambda i: (0, 0)),
        ],
        out_specs=pl.BlockSpec((_BN, DOUT), lambda i: (i, 0)),
        out_shape=jax.ShapeDtypeStruct((NPAD, DOUT), jnp.float32),
    )(agg2, h2p, deg_t, b2)


def kernel(x, edge_index, W1, b1, W2, b2):
    src = edge_index[0]
    dst = edge_index[1]

    pad = N + (jnp.arange(EPAD - E, dtype=src.dtype) % (NPAD - N))
    src2d = jnp.concatenate([src, pad]).reshape(EROWS, 128)
    dst2d = jnp.concatenate([dst, pad]).reshape(EROWS, 128)

    deg2 = _deg_partials(dst2d)
    deg_t = deg2.reshape(NC, NPAD, 128)[:, :, 0].T

    x_pad = jnp.zeros((NPAD, DIN), x.dtype).at[:N].set(x)

    xp = _tca(x_pad, deg_t)
    agg1 = _agg_edge(xp, src2d, dst2d).reshape(NC, NPAD, DIN)
    h2p = _tcb(agg1, xp, deg_t, b1.reshape(1, DHID), W1, W2)
    agg2 = _agg_edge(h2p, src2d, dst2d).reshape(NC, NPAD, DOUT)
    out = _tc3(agg2, h2p, deg_t, b2.reshape(1, DOUT))
    return out[:N]


# speedup vs baseline: 22.0415x; 1.0150x over previous
import functools

import jax
import jax.numpy as jnp
from jax import lax
from jax.experimental import pallas as pl
from jax.experimental.pallas import tpu as pltpu
from jax.experimental.pallas import tpu_sc as plsc

N = 10000
NPAD = 10240
E = 320000
DIN = 128
DHID = 256
DOUT = 128

NC = 2
NS = 16
L = 16
NW = NC * NS

EROWS = 2560
EPAD = EROWS * 128
RPT = EROWS // NS
NROWS_PT = NPAD // NS


def _sc_mesh():
    return plsc.VectorSubcoreMesh(
        core_axis_name="c", subcore_axis_name="s", num_cores=NC, num_subcores=NS
    )


_RPW = EROWS // NW


def _deg_partials(dst2d):
    @functools.partial(
        pl.kernel,
        mesh=_sc_mesh(),
        out_type=jax.ShapeDtypeStruct((NC * NPAD, 128), jnp.float32),
        scratch_types=[
            pltpu.VMEM((_RPW, 128), jnp.int32),
            pltpu.VMEM((128, 128), jnp.float32),
            pltpu.VMEM_SHARED((NPAD, 128), jnp.float32),
        ],
    )
    def deg_kernel(dst_hbm, out_hbm, di_v, rows_v, deg_sh):
        cid = lax.axis_index("c")
        sid = lax.axis_index("s")
        rbase = (cid * NS + sid) * _RPW
        obase = sid * NROWS_PT

        def zrow(i, _):
            def zvec(k, _):
                rows_v[i, pl.ds(k * L, L)] = jnp.zeros((L,), jnp.float32)
                return 0

            lax.fori_loop(0, 128 // L, zvec, 0)
            return 0

        lax.fori_loop(0, 128, zrow, 0)

        def clr(k, _):
            pltpu.sync_copy(rows_v, deg_sh.at[pl.ds(obase + k * 128, 128)])
            return 0

        lax.fori_loop(0, NROWS_PT // 128, clr, 0)
        plsc.subcore_barrier()

        def set1(i, _):
            def svec(k, _):
                rows_v[i, pl.ds(k * L, L)] = jnp.ones((L,), jnp.float32)
                return 0

            lax.fori_loop(0, 128 // L, svec, 0)
            return 0

        lax.fori_loop(0, 128, set1, 0)

        pltpu.sync_copy(dst_hbm.at[pl.ds(rbase, _RPW)], di_v)

        def main(j, _):
            pltpu.sync_copy(rows_v, deg_sh.at[di_v.at[j]], add=True)
            return 0

        lax.fori_loop(0, _RPW, main, 0)
        plsc.subcore_barrier()

        pltpu.sync_copy(deg_sh.at[pl.ds(obase, NROWS_PT)],
                        out_hbm.at[pl.ds(cid * NPAD + obase, NROWS_PT)])

    return deg_kernel(dst2d)


CHR = 40


_RPT_ES = EROWS // NW


def _agg_edge(h, src2d, dst2d):
    @functools.partial(
        pl.kernel,
        mesh=_sc_mesh(),
        out_type=jax.ShapeDtypeStruct((NC * NPAD, DOUT), jnp.float32),
        scratch_types=[
            pltpu.VMEM((CHR, 128), jnp.int32),
            pltpu.VMEM((CHR, 128), jnp.int32),
            pltpu.VMEM((128, DOUT), jnp.float32),
            pltpu.VMEM((128, DOUT), jnp.float32),
            pltpu.VMEM_SHARED((NPAD, DOUT), jnp.float32),
            pltpu.SemaphoreType.DMA,
            pltpu.SemaphoreType.DMA,
            pltpu.SemaphoreType.DMA,
            pltpu.SemaphoreType.DMA,
        ],
    )
    def agg_kernel(h_hbm, src_hbm, dst_hbm, out_hbm, si_v, di_v, rows_a, rows_b,
                   agg_sh, sem_a, sem_b, sem_sa, sem_sb):
        cid = lax.axis_index("c")
        sid = lax.axis_index("s")
        rbase = (cid * NS + sid) * _RPT_ES

        def zrow(i, _):
            def zvec(k, _):
                rows_a[i, pl.ds(k * L, L)] = jnp.zeros((L,), jnp.float32)
                return 0

            lax.fori_loop(0, DOUT // L, zvec, 0)
            return 0

        lax.fori_loop(0, 128, zrow, 0)

        def clr(k, _):
            pltpu.sync_copy(rows_a, agg_sh.at[pl.ds(sid * NROWS_PT + k * 128, 128)])
            return 0

        lax.fori_loop(0, NROWS_PT // 128, clr, 0)
        plsc.subcore_barrier()

        def chunk(q, _):
            rq = rbase + q * CHR
            pltpu.sync_copy(src_hbm.at[pl.ds(rq, CHR)], si_v)
            pltpu.sync_copy(dst_hbm.at[pl.ds(rq, CHR)], di_v)

            def pair(p, _):
                cpa = pltpu.async_copy(h_hbm.at[si_v.at[2 * p]], rows_a, sem_a)
                cpb = pltpu.async_copy(h_hbm.at[si_v.at[2 * p + 1]], rows_b, sem_b)
                cpa.wait()
                sca = pltpu.async_copy(rows_a, agg_sh.at[di_v.at[2 * p]],
                                       sem_sa, add=True)
                cpb.wait()
                scb = pltpu.async_copy(rows_b, agg_sh.at[di_v.at[2 * p + 1]],
                                       sem_sb, add=True)
                sca.wait()
                scb.wait()
                return 0

            lax.fori_loop(0, CHR // 2, pair, 0)
            return 0

        lax.fori_loop(0, _RPT_ES // CHR, chunk, 0)
        plsc.subcore_barrier()

        pltpu.sync_copy(
            agg_sh.at[pl.ds(sid * NROWS_PT, NROWS_PT)],
            out_hbm.at[pl.ds(cid * NPAD + sid * NROWS_PT, NROWS_PT)],
        )

    return agg_kernel(h, src2d, dst2d)


_BN = 512


def _dis(deg_ref):
    return lax.rsqrt(deg_ref[:, 0:1] + deg_ref[:, 1:2] + 1.0)


def _tca_body(x_ref, deg_ref, out_ref):
    out_ref[...] = _dis(deg_ref) * x_ref[...]


def _tca(x_pad, deg_t):
    return pl.pallas_call(
        _tca_body,
        grid=(NPAD // _BN,),
        in_specs=[
            pl.BlockSpec((_BN, DIN), lambda i: (i, 0)),
            pl.BlockSpec((_BN, 2), lambda i: (i, 0)),
        ],
        out_specs=pl.BlockSpec((_BN, DIN), lambda i: (i, 0)),
        out_shape=jax.ShapeDtypeStruct((NPAD, DIN), jnp.float32),
    )(x_pad, deg_t)


def _tcb_body(agg_ref, xp_ref, deg_ref, b1_ref, w1_ref, w2_ref, out_ref):
    dis = _dis(deg_ref)
    u = agg_ref[0] + agg_ref[1] + xp_ref[...]
    t = jnp.dot(u, w1_ref[...], preferred_element_type=jnp.float32)
    z = jnp.maximum(dis * t + b1_ref[...], 0.0)
    h2 = jnp.dot(z, w2_ref[...], preferred_element_type=jnp.float32)
    out_ref[...] = dis * h2


def _tcb(agg1, xp, deg_t, b1, w1, w2):
    return pl.pallas_call(
        _tcb_body,
        grid=(NPAD // _BN,),
        in_specs=[
            pl.BlockSpec((NC, _BN, DIN), lambda i: (0, i, 0)),
            pl.BlockSpec((_BN, DIN), lambda i: (i, 0)),
            pl.BlockSpec((_BN, 2), lambda i: (i, 0)),
            pl.BlockSpec((1, DHID), lambda i: (0, 0)),
            pl.BlockSpec((DIN, DHID), lambda i: (0, 0)),
            pl.BlockSpec((DHID, DOUT), lambda i: (0, 0)),
        ],
        out_specs=pl.BlockSpec((_BN, DOUT), lambda i: (i, 0)),
        out_shape=jax.ShapeDtypeStruct((NPAD, DOUT), jnp.float32),
    )(agg1, xp, deg_t, b1, w1, w2)


def _tc3_body(agg_ref, hp_ref, deg_ref, b2_ref, out_ref):
    dis = _dis(deg_ref)
    s = agg_ref[0] + agg_ref[1] + hp_ref[...]
    out_ref[...] = jnp.maximum(dis * s + b2_ref[...], 0.0)


def _tc3(agg2, h2p, deg_t, b2):
    return pl.pallas_call(
        _tc3_body,
        grid=(NPAD // _BN,),
        in_specs=[
            pl.BlockSpec((NC, _BN, DOUT), lambda i: (0, i, 0)),
            pl.BlockSpec((_BN, DOUT), lambda i: (i, 0)),
            pl.BlockSpec((_BN, 2), lambda i: (i, 0)),
            pl.BlockSpec((1, DOUT), lambda i: (0, 0)),
        ],
        out_specs=pl.BlockSpec((_BN, DOUT), lambda i: (i, 0)),
        out_shape=jax.ShapeDtypeStruct((NPAD, DOUT), jnp.float32),
    )(agg2, h2p, deg_t, b2)


def kernel(x, edge_index, W1, b1, W2, b2):
    src = edge_index[0]
    dst = edge_index[1]

    pad = N + (jnp.arange(EPAD - E, dtype=src.dtype) % (NPAD - N))
    src2d = jnp.concatenate([src, pad]).reshape(EROWS, 128)
    dst2d = jnp.concatenate([dst, pad]).reshape(EROWS, 128)

    deg2 = _deg_partials(dst2d)
    deg_t = deg2.reshape(NC, NPAD, 128)[:, :, 0].T

    x_pad = jnp.zeros((NPAD, DIN), x.dtype).at[:N].set(x)

    xp = _tca(x_pad, deg_t)
    agg1 = _agg_edge(xp, src2d, dst2d).reshape(NC, NPAD, DIN)
    h2p = _tcb(agg1, xp, deg_t, b1.reshape(1, DHID), W1, W2)
    agg2 = _agg_edge(h2p, src2d, dst2d).reshape(NC, NPAD, DOUT)
    out = _tc3(agg2, h2p, deg_t, b2.reshape(1, DOUT))
    return out[:N]


# speedup vs baseline: 25.0842x; 1.1380x over previous
import functools

import jax
import jax.numpy as jnp
from jax import lax
from jax.experimental import pallas as pl
from jax.experimental.pallas import tpu as pltpu
from jax.experimental.pallas import tpu_sc as plsc

N = 10000
NPAD = 10240
E = 320000
DIN = 128
DHID = 256
DOUT = 128

NC = 2
NS = 16
L = 16
NW = NC * NS

EROWS = 2560
EPAD = EROWS * 128
RPT = EROWS // NS
NROWS_PT = NPAD // NS


def _sc_mesh():
    return plsc.VectorSubcoreMesh(
        core_axis_name="c", subcore_axis_name="s", num_cores=NC, num_subcores=NS
    )


_RPW = EROWS // NW


def _deg_partials(dst2d):
    @functools.partial(
        pl.kernel,
        mesh=_sc_mesh(),
        out_type=jax.ShapeDtypeStruct((NC * NPAD, 128), jnp.float32),
        scratch_types=[
            pltpu.VMEM((_RPW, 128), jnp.int32),
            pltpu.VMEM((128, 128), jnp.float32),
            pltpu.VMEM_SHARED((NPAD, 128), jnp.float32),
        ],
    )
    def deg_kernel(dst_hbm, out_hbm, di_v, rows_v, deg_sh):
        cid = lax.axis_index("c")
        sid = lax.axis_index("s")
        rbase = (cid * NS + sid) * _RPW
        obase = sid * NROWS_PT

        def zrow(i, _):
            def zvec(k, _):
                rows_v[i, pl.ds(k * L, L)] = jnp.zeros((L,), jnp.float32)
                return 0

            lax.fori_loop(0, 128 // L, zvec, 0)
            return 0

        lax.fori_loop(0, 128, zrow, 0)

        def clr(k, _):
            pltpu.sync_copy(rows_v, deg_sh.at[pl.ds(obase + k * 128, 128)])
            return 0

        lax.fori_loop(0, NROWS_PT // 128, clr, 0)
        plsc.subcore_barrier()

        def set1(i, _):
            def svec(k, _):
                rows_v[i, pl.ds(k * L, L)] = jnp.ones((L,), jnp.float32)
                return 0

            lax.fori_loop(0, 128 // L, svec, 0)
            return 0

        lax.fori_loop(0, 128, set1, 0)

        pltpu.sync_copy(dst_hbm.at[pl.ds(rbase, _RPW)], di_v)

        def main(j, _):
            pltpu.sync_copy(rows_v, deg_sh.at[di_v.at[j]], add=True)
            return 0

        lax.fori_loop(0, _RPW, main, 0)
        plsc.subcore_barrier()

        pltpu.sync_copy(deg_sh.at[pl.ds(obase, NROWS_PT)],
                        out_hbm.at[pl.ds(cid * NPAD + obase, NROWS_PT)])

    return deg_kernel(dst2d)


CHR = 40


_RPT_ES = EROWS // NW


def _agg_edge(h, src2d, dst2d):
    @functools.partial(
        pl.kernel,
        mesh=_sc_mesh(),
        out_type=jax.ShapeDtypeStruct((NC * NPAD, DOUT), jnp.float32),
        scratch_types=[
            pltpu.VMEM((CHR, 128), jnp.int32),
            pltpu.VMEM((CHR, 128), jnp.int32),
            pltpu.VMEM((128, DOUT), jnp.float32),
            pltpu.VMEM((128, DOUT), jnp.float32),
            pltpu.VMEM_SHARED((NPAD, DOUT), jnp.float32),
            pltpu.SemaphoreType.DMA,
            pltpu.SemaphoreType.DMA,
            pltpu.SemaphoreType.DMA,
            pltpu.SemaphoreType.DMA,
        ],
    )
    def agg_kernel(h_hbm, src_hbm, dst_hbm, out_hbm, si_v, di_v, rows_a, rows_b,
                   agg_sh, sem_a, sem_b, sem_sa, sem_sb):
        cid = lax.axis_index("c")
        sid = lax.axis_index("s")
        rbase = (cid * NS + sid) * _RPT_ES

        def zrow(i, _):
            def zvec(k, _):
                rows_a[i, pl.ds(k * L, L)] = jnp.zeros((L,), jnp.float32)
                return 0

            lax.fori_loop(0, DOUT // L, zvec, 0)
            return 0

        lax.fori_loop(0, 128, zrow, 0)

        def clr(k, _):
            pltpu.sync_copy(rows_a, agg_sh.at[pl.ds(sid * NROWS_PT + k * 128, 128)])
            return 0

        lax.fori_loop(0, NROWS_PT // 128, clr, 0)
        plsc.subcore_barrier()

        def chunk(q, _):
            rq = rbase + q * CHR
            pltpu.sync_copy(src_hbm.at[pl.ds(rq, CHR)], si_v)
            pltpu.sync_copy(dst_hbm.at[pl.ds(rq, CHR)], di_v)

            def pair(p, _):
                cpa = pltpu.async_copy(h_hbm.at[si_v.at[2 * p]], rows_a, sem_a)
                cpb = pltpu.async_copy(h_hbm.at[si_v.at[2 * p + 1]], rows_b, sem_b)
                cpa.wait()
                sca = pltpu.async_copy(rows_a, agg_sh.at[di_v.at[2 * p]],
                                       sem_sa, add=True)
                cpb.wait()
                scb = pltpu.async_copy(rows_b, agg_sh.at[di_v.at[2 * p + 1]],
                                       sem_sb, add=True)
                sca.wait()
                scb.wait()
                return 0

            lax.fori_loop(0, CHR // 2, pair, 0)
            return 0

        lax.fori_loop(0, _RPT_ES // CHR, chunk, 0)
        plsc.subcore_barrier()

        pltpu.sync_copy(
            agg_sh.at[pl.ds(sid * NROWS_PT, NROWS_PT)],
            out_hbm.at[pl.ds(cid * NPAD + sid * NROWS_PT, NROWS_PT)],
        )

    return agg_kernel(h, src2d, dst2d)


_BN = 512


def _tca_body(x_ref, dega_ref, degb_ref, xp_ref, dis_ref):
    dis = lax.rsqrt(dega_ref[:, 0:1] + degb_ref[:, 0:1] + 1.0)
    dis_ref[...] = dis
    xp_ref[...] = dis * x_ref[...]


def _tca(x_pad, deg2):
    return pl.pallas_call(
        _tca_body,
        grid=(NPAD // _BN,),
        in_specs=[
            pl.BlockSpec((_BN, DIN), lambda i: (i, 0)),
            pl.BlockSpec((_BN, 128), lambda i: (i, 0)),
            pl.BlockSpec((_BN, 128), lambda i: (i + NPAD // _BN, 0)),
        ],
        out_specs=[
            pl.BlockSpec((_BN, DIN), lambda i: (i, 0)),
            pl.BlockSpec((_BN, 1), lambda i: (i, 0)),
        ],
        out_shape=[
            jax.ShapeDtypeStruct((NPAD, DIN), jnp.float32),
            jax.ShapeDtypeStruct((NPAD, 1), jnp.float32),
        ],
    )(x_pad, deg2, deg2)


def _tcb_body(agg_ref, xp_ref, dis_ref, b1_ref, w1_ref, w2_ref, out_ref):
    dis = dis_ref[...]
    u = agg_ref[0] + agg_ref[1] + xp_ref[...]
    t = jnp.dot(u, w1_ref[...], preferred_element_type=jnp.float32)
    z = jnp.maximum(dis * t + b1_ref[...], 0.0)
    h2 = jnp.dot(z, w2_ref[...], preferred_element_type=jnp.float32)
    out_ref[...] = dis * h2


def _tcb(agg1, xp, dis, b1, w1, w2):
    return pl.pallas_call(
        _tcb_body,
        grid=(NPAD // _BN,),
        in_specs=[
            pl.BlockSpec((NC, _BN, DIN), lambda i: (0, i, 0)),
            pl.BlockSpec((_BN, DIN), lambda i: (i, 0)),
            pl.BlockSpec((_BN, 1), lambda i: (i, 0)),
            pl.BlockSpec((1, DHID), lambda i: (0, 0)),
            pl.BlockSpec((DIN, DHID), lambda i: (0, 0)),
            pl.BlockSpec((DHID, DOUT), lambda i: (0, 0)),
        ],
        out_specs=pl.BlockSpec((_BN, DOUT), lambda i: (i, 0)),
        out_shape=jax.ShapeDtypeStruct((NPAD, DOUT), jnp.float32),
    )(agg1, xp, dis, b1, w1, w2)


def _tc3_body(agg_ref, hp_ref, dis_ref, b2_ref, out_ref):
    dis = dis_ref[...]
    s = agg_ref[0] + agg_ref[1] + hp_ref[...]
    out_ref[...] = jnp.maximum(dis * s + b2_ref[...], 0.0)


def _tc3(agg2, h2p, dis, b2):
    return pl.pallas_call(
        _tc3_body,
        grid=(NPAD // _BN,),
        in_specs=[
            pl.BlockSpec((NC, _BN, DOUT), lambda i: (0, i, 0)),
            pl.BlockSpec((_BN, DOUT), lambda i: (i, 0)),
            pl.BlockSpec((_BN, 1), lambda i: (i, 0)),
            pl.BlockSpec((1, DOUT), lambda i: (0, 0)),
        ],
        out_specs=pl.BlockSpec((_BN, DOUT), lambda i: (i, 0)),
        out_shape=jax.ShapeDtypeStruct((NPAD, DOUT), jnp.float32),
    )(agg2, h2p, dis, b2)


def kernel(x, edge_index, W1, b1, W2, b2):
    src = edge_index[0]
    dst = edge_index[1]

    pad = N + (jnp.arange(EPAD - E, dtype=src.dtype) % (NPAD - N))
    src2d = jnp.concatenate([src, pad]).reshape(EROWS, 128)
    dst2d = jnp.concatenate([dst, pad]).reshape(EROWS, 128)

    deg2 = _deg_partials(dst2d)

    x_pad = jnp.zeros((NPAD, DIN), x.dtype).at[:N].set(x)

    xp, dis = _tca(x_pad, deg2)
    agg1 = _agg_edge(xp, src2d, dst2d).reshape(NC, NPAD, DIN)
    h2p = _tcb(agg1, xp, dis, b1.reshape(1, DHID), W1, W2)
    agg2 = _agg_edge(h2p, src2d, dst2d).reshape(NC, NPAD, DOUT)
    out = _tc3(agg2, h2p, dis, b2.reshape(1, DOUT))
    return out[:N]


# speedup vs baseline: 25.1945x; 1.0044x over previous
import functools

import jax
import jax.numpy as jnp
from jax import lax
from jax.experimental import pallas as pl
from jax.experimental.pallas import tpu as pltpu
from jax.experimental.pallas import tpu_sc as plsc

N = 10000
NPAD = 10240
E = 320000
DIN = 128
DHID = 256
DOUT = 128

NC = 2
NS = 16
L = 16
NW = NC * NS

EROWS = 2560
EPAD = EROWS * 128
RPT = EROWS // NS
NROWS_PT = NPAD // NS


def _sc_mesh():
    return plsc.VectorSubcoreMesh(
        core_axis_name="c", subcore_axis_name="s", num_cores=NC, num_subcores=NS
    )


_RPW = EROWS // NW


def _deg_partials(dst2d):
    @functools.partial(
        pl.kernel,
        mesh=_sc_mesh(),
        out_type=jax.ShapeDtypeStruct((NC * NPAD, 128), jnp.float32),
        scratch_types=[
            pltpu.VMEM((_RPW, 128), jnp.int32),
            pltpu.VMEM((128, 128), jnp.float32),
            pltpu.VMEM_SHARED((NPAD, 128), jnp.float32),
        ],
    )
    def deg_kernel(dst_hbm, out_hbm, di_v, rows_v, deg_sh):
        cid = lax.axis_index("c")
        sid = lax.axis_index("s")
        rbase = (cid * NS + sid) * _RPW
        obase = sid * NROWS_PT

        def zrow(i, _):
            def zvec(k, _):
                rows_v[i, pl.ds(k * L, L)] = jnp.zeros((L,), jnp.float32)
                return 0

            lax.fori_loop(0, 128 // L, zvec, 0)
            return 0

        lax.fori_loop(0, 128, zrow, 0)

        def clr(k, _):
            pltpu.sync_copy(rows_v, deg_sh.at[pl.ds(obase + k * 128, 128)])
            return 0

        lax.fori_loop(0, NROWS_PT // 128, clr, 0)
        plsc.subcore_barrier()

        def set1(i, _):
            def svec(k, _):
                rows_v[i, pl.ds(k * L, L)] = jnp.ones((L,), jnp.float32)
                return 0

            lax.fori_loop(0, 128 // L, svec, 0)
            return 0

        lax.fori_loop(0, 128, set1, 0)

        pltpu.sync_copy(dst_hbm.at[pl.ds(rbase, _RPW)], di_v)

        def main(j, _):
            pltpu.sync_copy(rows_v, deg_sh.at[di_v.at[j]], add=True)
            return 0

        lax.fori_loop(0, _RPW, main, 0)
        plsc.subcore_barrier()

        pltpu.sync_copy(deg_sh.at[pl.ds(obase, NROWS_PT)],
                        out_hbm.at[pl.ds(cid * NPAD + obase, NROWS_PT)])

    return deg_kernel(dst2d)


CHR = 40


_RPT_ES = EROWS // NW


def _agg_edge(h, src2d, dst2d):
    @functools.partial(
        pl.kernel,
        mesh=_sc_mesh(),
        out_type=jax.ShapeDtypeStruct((NC * NPAD, DOUT), jnp.float32),
        scratch_types=[
            pltpu.VMEM((CHR, 128), jnp.int32),
            pltpu.VMEM((CHR, 128), jnp.int32),
            pltpu.VMEM((128, DOUT), jnp.float32),
            pltpu.VMEM((128, DOUT), jnp.float32),
            pltpu.VMEM_SHARED((NPAD, DOUT), jnp.float32),
            pltpu.SemaphoreType.DMA,
            pltpu.SemaphoreType.DMA,
            pltpu.SemaphoreType.DMA,
            pltpu.SemaphoreType.DMA,
        ],
    )
    def agg_kernel(h_hbm, src_hbm, dst_hbm, out_hbm, si_v, di_v, rows_a, rows_b,
                   agg_sh, sem_a, sem_b, sem_sa, sem_sb):
        cid = lax.axis_index("c")
        sid = lax.axis_index("s")
        rbase = (cid * NS + sid) * _RPT_ES

        def zrow(i, _):
            def zvec(k, _):
                rows_a[i, pl.ds(k * L, L)] = jnp.zeros((L,), jnp.float32)
                return 0

            lax.fori_loop(0, DOUT // L, zvec, 0)
            return 0

        lax.fori_loop(0, 128, zrow, 0)

        def clr(k, _):
            pltpu.sync_copy(rows_a, agg_sh.at[pl.ds(sid * NROWS_PT + k * 128, 128)])
            return 0

        lax.fori_loop(0, NROWS_PT // 128, clr, 0)
        plsc.subcore_barrier()

        def chunk(q, _):
            rq = rbase + q * CHR
            pltpu.sync_copy(src_hbm.at[pl.ds(rq, CHR)], si_v)
            pltpu.sync_copy(dst_hbm.at[pl.ds(rq, CHR)], di_v)

            def pair(p, _):
                cpa = pltpu.async_copy(h_hbm.at[si_v.at[2 * p]], rows_a, sem_a)
                cpb = pltpu.async_copy(h_hbm.at[si_v.at[2 * p + 1]], rows_b, sem_b)
                cpa.wait()
                sca = pltpu.async_copy(rows_a, agg_sh.at[di_v.at[2 * p]],
                                       sem_sa, add=True)
                cpb.wait()
                scb = pltpu.async_copy(rows_b, agg_sh.at[di_v.at[2 * p + 1]],
                                       sem_sb, add=True)
                sca.wait()
                scb.wait()
                return 0

            lax.fori_loop(0, CHR // 2, pair, 0)
            return 0

        lax.fori_loop(0, _RPT_ES // CHR, chunk, 0)
        plsc.subcore_barrier()

        pltpu.sync_copy(
            agg_sh.at[pl.ds(sid * NROWS_PT, NROWS_PT)],
            out_hbm.at[pl.ds(cid * NPAD + sid * NROWS_PT, NROWS_PT)],
        )

    return agg_kernel(h, src2d, dst2d)


_BN = 512


def _tca_body(x_ref, dega_ref, degb_ref, xp_ref, dis_ref):
    dis = lax.rsqrt(dega_ref[:, 0:1] + degb_ref[:, 0:1] + 1.0)
    dis_ref[...] = dis
    xp_ref[...] = dis * x_ref[...]


def _tca(x_pad, deg2):
    return pl.pallas_call(
        _tca_body,
        grid=(NPAD // _BN,),
        in_specs=[
            pl.BlockSpec((_BN, DIN), lambda i: (i, 0)),
            pl.BlockSpec((_BN, 128), lambda i: (i, 0)),
            pl.BlockSpec((_BN, 128), lambda i: (i + NPAD // _BN, 0)),
        ],
        out_specs=[
            pl.BlockSpec((_BN, DIN), lambda i: (i, 0)),
            pl.BlockSpec((_BN, 1), lambda i: (i, 0)),
        ],
        out_shape=[
            jax.ShapeDtypeStruct((NPAD, DIN), jnp.float32),
            jax.ShapeDtypeStruct((NPAD, 1), jnp.float32),
        ],
    )(x_pad, deg2, deg2)


def _tcb_body(agg_ref, xp_ref, dis_ref, b1_ref, w1_ref, w2_ref, out_ref):
    dis = dis_ref[...]
    u = agg_ref[0] + agg_ref[1] + xp_ref[...]
    t = jnp.dot(u, w1_ref[...], preferred_element_type=jnp.float32)
    z = jnp.maximum(dis * t + b1_ref[...], 0.0)
    h2 = jnp.dot(z, w2_ref[...], preferred_element_type=jnp.float32)
    out_ref[...] = dis * h2


def _tcb(agg1, xp, dis, b1, w1, w2):
    return pl.pallas_call(
        _tcb_body,
        grid=(NPAD // _BN,),
        in_specs=[
            pl.BlockSpec((NC, _BN, DIN), lambda i: (0, i, 0)),
            pl.BlockSpec((_BN, DIN), lambda i: (i, 0)),
            pl.BlockSpec((_BN, 1), lambda i: (i, 0)),
            pl.BlockSpec((1, DHID), lambda i: (0, 0)),
            pl.BlockSpec((DIN, DHID), lambda i: (0, 0)),
            pl.BlockSpec((DHID, DOUT), lambda i: (0, 0)),
        ],
        out_specs=pl.BlockSpec((_BN, DOUT), lambda i: (i, 0)),
        out_shape=jax.ShapeDtypeStruct((NPAD, DOUT), jnp.float32),
    )(agg1, xp, dis, b1, w1, w2)


def _tc3_body(agg_ref, hp_ref, dis_ref, b2_ref, out_ref):
    dis = dis_ref[...]
    s = agg_ref[0] + agg_ref[1] + hp_ref[...]
    out_ref[...] = jnp.maximum(dis * s + b2_ref[...], 0.0)


_BN3 = 400


def _tc3(agg2, h2p, dis, b2):
    return pl.pallas_call(
        _tc3_body,
        grid=(N // _BN3,),
        in_specs=[
            pl.BlockSpec((NC, _BN3, DOUT), lambda i: (0, i, 0)),
            pl.BlockSpec((_BN3, DOUT), lambda i: (i, 0)),
            pl.BlockSpec((_BN3, 1), lambda i: (i, 0)),
            pl.BlockSpec((1, DOUT), lambda i: (0, 0)),
        ],
        out_specs=pl.BlockSpec((_BN3, DOUT), lambda i: (i, 0)),
        out_shape=jax.ShapeDtypeStruct((N, DOUT), jnp.float32),
    )(agg2, h2p, dis, b2)


def kernel(x, edge_index, W1, b1, W2, b2):
    src = edge_index[0]
    dst = edge_index[1]

    pad = N + (jnp.arange(EPAD - E, dtype=src.dtype) % (NPAD - N))
    src2d = jnp.concatenate([src, pad]).reshape(EROWS, 128)
    dst2d = jnp.concatenate([dst, pad]).reshape(EROWS, 128)

    deg2 = _deg_partials(dst2d)

    x_pad = jnp.zeros((NPAD, DIN), x.dtype).at[:N].set(x)

    xp, dis = _tca(x_pad, deg2)
    agg1 = _agg_edge(xp, src2d, dst2d).reshape(NC, NPAD, DIN)
    h2p = _tcb(agg1, xp, dis, b1.reshape(1, DHID), W1, W2)
    agg2 = _agg_edge(h2p, src2d, dst2d).reshape(NC, NPAD, DOUT)
    return _tc3(agg2, h2p, dis, b2.reshape(1, DOUT))
